# baseline (speedup 1.0000x reference)
import jax
import jax.numpy as jnp
from jax import lax
from jax.experimental import pallas as pl
from jax.experimental.pallas import tpu as pltpu
from jax.experimental.pallas import tpu_sc as plsc

NC = 2
NS = 16
CHUNK = 128

_N = 10000
_D = 128
_HD = _D // 2
_N_ACC = 10240
_ZROWS = _N_ACC // NS
_DROWS = 128


def _mesh():
    return plsc.VectorSubcoreMesh(core_axis_name="c", subcore_axis_name="s")


def _sc_deg(c3, z128, identp, nch):
    half = nch // 2

    def body(c_hbm, z_hbm, id_hbm, out0, out1, cidx, buf, ident, hist, acc):
        ones16 = jnp.full((16,), 1.0, jnp.float32)
        cid = lax.axis_index("c")
        sid = lax.axis_index("s")
        pltpu.sync_copy(z_hbm, buf)
        pltpu.sync_copy(z_hbm, hist)

        @pl.when(sid == 0)
        def _():
            pltpu.sync_copy(buf, acc)

        pltpu.sync_copy(id_hbm, ident)
        pltpu.sync_copy(c_hbm.at[sid, pl.ds(cid * half, half)], cidx)
        plsc.subcore_barrier()

        def step(k, _):
            for j in range(CHUNK // 16):
                c = cidx[k, pl.ds(j * 16, 16)]
                row = lax.shift_right_logical(c, 7)
                lane = lax.bitwise_and(c, 127)
                plsc.addupdate_scatter(hist, [row, lane], ones16)
            return _

        lax.fori_loop(0, half, step, None)
        pltpu.sync_copy(hist, acc.at[ident.at[0]], add=True)
        plsc.subcore_barrier()

        def copy_out(dst):
            pltpu.sync_copy(acc, buf)
            pltpu.sync_copy(buf, dst)

        @pl.when((sid == 0) & (cid == 0))
        def _():
            copy_out(out0)

        @pl.when((sid == 0) & (cid == 1))
        def _():
            copy_out(out1)

    sds = jax.ShapeDtypeStruct((_DROWS, 128), jnp.float32)
    return pl.kernel(
        body,
        out_type=(sds, sds),
        mesh=_mesh(),
        compiler_params=pltpu.CompilerParams(needs_layout_passes=False),
        scratch_types=[
            pltpu.VMEM((half, CHUNK), jnp.int32),
            pltpu.VMEM((_DROWS, 128), jnp.float32),
            pltpu.VMEM((8, CHUNK), jnp.int32),
            pltpu.VMEM((_DROWS, 128), jnp.float32),
            pltpu.VMEM_SHARED((_DROWS, 128), jnp.float32),
        ],
    )(c3, z128, identp)


_NBUF = 4


def _sc_aggregate(g2, r3, c3, z64, nch):
    def body(g_hbm, r_hbm, c_hbm, z_hbm, out,
             ridx, cidx, bufs, acc, gsems, ssems):
        cid = lax.axis_index("c")
        sid = lax.axis_index("s")
        gsrc = g_hbm.at[cid]
        pltpu.sync_copy(z_hbm, bufs.at[0])
        for j in range(_ZROWS // 128):
            pltpu.sync_copy(bufs.at[0, pl.ds(0, 128)],
                            acc.at[pl.ds(sid * _ZROWS + j * 128, 128)])
        pltpu.sync_copy(r_hbm.at[sid], ridx)
        pltpu.sync_copy(c_hbm.at[sid], cidx)
        plsc.subcore_barrier()

        def gather(k, b):
            return pltpu.make_async_copy(
                gsrc.at[ridx.at[k]], bufs.at[b], gsems.at[b])

        def scatter(k, b):
            return pltpu.make_async_copy(
                bufs.at[b], acc.at[cidx.at[k]], ssems.at[b])

        for b in range(_NBUF):
            gather(b, b).start()

        def step(i, _):
            k = _NBUF * i
            for b in range(_NBUF):
                gather(k + b, b).wait()
                scatter(k + b, b).start(add=True)
            for b in range(_NBUF):
                @pl.when(k + b + _NBUF < nch)
                def _():
                    scatter(k + b, b).wait()
                    gather(k + b + _NBUF, b).start()
            return _

        lax.fori_loop(0, nch // _NBUF, step, None)
        for b in range(_NBUF):
            scatter(nch - _NBUF + b, b).wait()
        plsc.subcore_barrier()

        for j in range(_ZROWS // 128):
            row = sid * _ZROWS + j * 128
            pltpu.sync_copy(acc.at[pl.ds(row, 128)], bufs.at[0, pl.ds(0, 128)])
            pltpu.sync_copy(bufs.at[0, pl.ds(0, 128)],
                            out.at[cid, pl.ds(row, 128)])

    return pl.kernel(
        body,
        out_type=jax.ShapeDtypeStruct((NC, _N_ACC, _HD), jnp.float32),
        mesh=_mesh(),
        compiler_params=pltpu.CompilerParams(use_tc_tiling_on_sc=False),
        scratch_types=[
            pltpu.VMEM((nch, CHUNK), jnp.int32),
            pltpu.VMEM((nch, CHUNK), jnp.int32),
            pltpu.VMEM((_NBUF, CHUNK, _HD), jnp.float32),
            pltpu.VMEM_SHARED((_N_ACC, _HD), jnp.float32),
            pltpu.SemaphoreType.DMA((_NBUF,)),
            pltpu.SemaphoreType.DMA((_NBUF,)),
        ],
    )(g2, r3, c3, z64)


_BN = 1000


def _tc_first(deg0, deg1, x, w1):
    def body(d0_ref, d1_ref, x_ref, w_ref, dis_ref, g_ref):
        deg = d0_ref[...] + d1_ref[...] + 1.0
        dis = lax.rsqrt(deg)
        dis_ref[...] = dis
        h = jnp.dot(x_ref[...], w_ref[...], preferred_element_type=jnp.float32)
        g = h * dis
        g_ref[0, :, :] = g[:, :_HD]
        g_ref[1, :, :] = g[:, _HD:]

    grid = (_N // _BN,)
    return pl.pallas_call(
        body,
        grid=grid,
        in_specs=[
            pl.BlockSpec((_BN, 1), lambda i: (i, 0)),
            pl.BlockSpec((_BN, 1), lambda i: (i, 0)),
            pl.BlockSpec((_BN, _D), lambda i: (i, 0)),
            pl.BlockSpec((_D, _D), lambda i: (0, 0)),
        ],
        out_specs=[
            pl.BlockSpec((_BN, 1), lambda i: (i, 0)),
            pl.BlockSpec((NC, _BN, _HD), lambda i: (0, i, 0)),
        ],
        out_shape=[
            jax.ShapeDtypeStruct((_N, 1), jnp.float32),
            jax.ShapeDtypeStruct((NC, _N, _HD), jnp.float32),
        ],
    )(deg0, deg1, x, w1)


def _tc_mid(dis, s, g, w2, b1):
    def body(dis_ref, s_ref, g_ref, w_ref, b_ref, g2_ref):
        dis = dis_ref[...]
        agg = jnp.concatenate([s_ref[0] + g_ref[0], s_ref[1] + g_ref[1]],
                              axis=1)
        x2 = dis * agg + b_ref[...]
        h2 = jnp.dot(x2, w_ref[...], preferred_element_type=jnp.float32)
        g2 = h2 * dis
        g2_ref[0, :, :] = g2[:, :_HD]
        g2_ref[1, :, :] = g2[:, _HD:]

    grid = (_N // _BN,)
    return pl.pallas_call(
        body,
        grid=grid,
        in_specs=[
            pl.BlockSpec((_BN, 1), lambda i: (i, 0)),
            pl.BlockSpec((NC, _BN, _HD), lambda i: (0, i, 0)),
            pl.BlockSpec((NC, _BN, _HD), lambda i: (0, i, 0)),
            pl.BlockSpec((_D, _D), lambda i: (0, 0)),
            pl.BlockSpec((1, _D), lambda i: (0, 0)),
        ],
        out_specs=[pl.BlockSpec((NC, _BN, _HD), lambda i: (0, i, 0))],
        out_shape=[jax.ShapeDtypeStruct((NC, _N, _HD), jnp.float32)],
    )(dis, s, g, w2, b1)[0]


def _tc_last(dis, s, g2, b2):
    def body(dis_ref, s_ref, g_ref, b_ref, o_ref):
        dis = dis_ref[...]
        agg = jnp.concatenate([s_ref[0] + g_ref[0], s_ref[1] + g_ref[1]],
                              axis=1)
        o_ref[...] = dis * agg + b_ref[...]

    grid = (_N // _BN,)
    return pl.pallas_call(
        body,
        grid=grid,
        in_specs=[
            pl.BlockSpec((_BN, 1), lambda i: (i, 0)),
            pl.BlockSpec((NC, _BN, _HD), lambda i: (0, i, 0)),
            pl.BlockSpec((NC, _BN, _HD), lambda i: (0, i, 0)),
            pl.BlockSpec((1, _D), lambda i: (0, 0)),
        ],
        out_specs=[pl.BlockSpec((_BN, _D), lambda i: (i, 0))],
        out_shape=[jax.ShapeDtypeStruct((_N, _D), jnp.float32)],
    )(dis, s, g2, b2)[0]


def kernel(x, edge_index, W1, b1, W2, b2):
    n, d = x.shape
    e = edge_index.shape[1]
    per = NS * CHUNK
    nch = 16 * (-(-e // (16 * per)))
    e_pad = nch * per
    row = edge_index[0]
    col = edge_index[1]
    pad = e_pad - e
    if pad:
        row = jnp.concatenate([row, jnp.zeros((pad,), jnp.int32)])
        col = jnp.concatenate([col, jnp.full((pad,), n, jnp.int32)])
    r3 = row.reshape(NS, nch, CHUNK)
    c3 = col.reshape(NS, nch, CHUNK)

    z64 = jnp.zeros((128, _HD), jnp.float32)
    z128 = jnp.zeros((128, 128), jnp.float32)
    identp = jnp.broadcast_to(jnp.arange(128, dtype=jnp.int32), (8, 128))
    b1r = b1.reshape(1, d)
    b2r = b2.reshape(1, d)

    deg0, deg1 = _sc_deg(c3, z128, identp, nch)
    deg0 = deg0.reshape(_DROWS * 128, 1)[:_N]
    deg1 = deg1.reshape(_DROWS * 128, 1)[:_N]
    dis, g1 = _tc_first(deg0, deg1, x, W1)
    s1 = _sc_aggregate(g1, r3, c3, z64, nch)
    g2 = _tc_mid(dis, s1, g1, W2, b1r)
    s2 = _sc_aggregate(g2, r3, c3, z64, nch)
    return _tc_last(dis, s2, g2, b2r)


# speedup vs baseline: 1.0006x; 1.0006x over previous
import jax
import jax.numpy as jnp
from jax import lax
from jax.experimental import pallas as pl
from jax.experimental.pallas import tpu as pltpu
from jax.experimental.pallas import tpu_sc as plsc

NC = 2
NS = 16
CHUNK = 128

_N = 10000
_D = 128
_HD = _D // 2
_N_ACC = 10240
_ZROWS = _N_ACC // NS
_DROWS = 128


def _mesh():
    return plsc.VectorSubcoreMesh(core_axis_name="c", subcore_axis_name="s")


def _sc_deg(c3, z128, identp, nch):
    half = nch // 2

    def body(c_hbm, z_hbm, id_hbm, out0, out1, cidx, buf, ident, hist, acc):
        ones16 = jnp.full((16,), 1.0, jnp.float32)
        cid = lax.axis_index("c")
        sid = lax.axis_index("s")
        pltpu.sync_copy(z_hbm, buf)
        pltpu.sync_copy(z_hbm, hist)

        @pl.when(sid == 0)
        def _():
            pltpu.sync_copy(buf, acc)

        pltpu.sync_copy(id_hbm, ident)
        pltpu.sync_copy(c_hbm.at[sid, pl.ds(cid * half, half)], cidx)
        plsc.subcore_barrier()

        def step(k, _):
            for j in range(CHUNK // 16):
                c = cidx[k, pl.ds(j * 16, 16)]
                row = lax.shift_right_logical(c, 7)
                lane = lax.bitwise_and(c, 127)
                plsc.addupdate_scatter(hist, [row, lane], ones16)
            return _

        lax.fori_loop(0, half, step, None)
        pltpu.sync_copy(hist, acc.at[ident.at[0]], add=True)
        plsc.subcore_barrier()

        def copy_out(dst):
            pltpu.sync_copy(acc, buf)
            pltpu.sync_copy(buf, dst)

        @pl.when((sid == 0) & (cid == 0))
        def _():
            copy_out(out0)

        @pl.when((sid == 0) & (cid == 1))
        def _():
            copy_out(out1)

    sds = jax.ShapeDtypeStruct((_DROWS, 128), jnp.float32)
    return pl.kernel(
        body,
        out_type=(sds, sds),
        mesh=_mesh(),
        compiler_params=pltpu.CompilerParams(needs_layout_passes=False),
        scratch_types=[
            pltpu.VMEM((half, CHUNK), jnp.int32),
            pltpu.VMEM((_DROWS, 128), jnp.float32),
            pltpu.VMEM((8, CHUNK), jnp.int32),
            pltpu.VMEM((_DROWS, 128), jnp.float32),
            pltpu.VMEM_SHARED((_DROWS, 128), jnp.float32),
        ],
    )(c3, z128, identp)


_NBUF = 4


def _sc_aggregate(g2, r3, c3, z64, nch):
    def body(g_hbm, r_hbm, c_hbm, z_hbm, out,
             ridx, cidx, bufs, acc, gsems, ssems):
        cid = lax.axis_index("c")
        sid = lax.axis_index("s")
        gsrc = g_hbm.at[cid]
        pltpu.sync_copy(z_hbm, bufs.at[0])
        for j in range(_ZROWS // 128):
            pltpu.sync_copy(bufs.at[0, pl.ds(0, 128)],
                            acc.at[pl.ds(sid * _ZROWS + j * 128, 128)])
        pltpu.sync_copy(r_hbm.at[sid], ridx)
        pltpu.sync_copy(c_hbm.at[sid], cidx)
        plsc.subcore_barrier()

        def gather(k, b):
            return pltpu.make_async_copy(
                gsrc.at[ridx.at[k]], bufs.at[b], gsems.at[b])

        def scatter(k, b):
            return pltpu.make_async_copy(
                bufs.at[b], acc.at[cidx.at[k]], ssems.at[b])

        for b in range(_NBUF):
            gather(b, b).start()

        def step(i, _):
            k = _NBUF * i
            for b in range(_NBUF):
                gather(k + b, b).wait()
                scatter(k + b, b).start(add=True)
            for b in range(_NBUF):
                @pl.when(k + b + _NBUF < nch)
                def _():
                    scatter(k + b, b).wait()
                    gather(k + b + _NBUF, b).start()
            return _

        lax.fori_loop(0, nch // _NBUF, step, None)
        for b in range(_NBUF):
            scatter(nch - _NBUF + b, b).wait()
        plsc.subcore_barrier()

        for j in range(_ZROWS // 128):
            row = sid * _ZROWS + j * 128
            pltpu.sync_copy(acc.at[pl.ds(row, 128)], bufs.at[0, pl.ds(0, 128)])
            pltpu.sync_copy(bufs.at[0, pl.ds(0, 128)],
                            out.at[cid, pl.ds(row, 128)])

    return pl.kernel(
        body,
        out_type=jax.ShapeDtypeStruct((NC, _N_ACC, _HD), jnp.float32),
        mesh=_mesh(),
        compiler_params=pltpu.CompilerParams(use_tc_tiling_on_sc=False),
        scratch_types=[
            pltpu.VMEM((nch, CHUNK), jnp.int32),
            pltpu.VMEM((nch, CHUNK), jnp.int32),
            pltpu.VMEM((_NBUF, CHUNK, _HD), jnp.float32),
            pltpu.VMEM_SHARED((_N_ACC, _HD), jnp.float32),
            pltpu.SemaphoreType.DMA((_NBUF,)),
            pltpu.SemaphoreType.DMA((_NBUF,)),
        ],
    )(g2, r3, c3, z64)


_BN = 1000


def _tc_matmul(x, w1):
    def body(x_ref, w_ref, h_ref):
        h_ref[...] = jnp.dot(x_ref[...], w_ref[...],
                             preferred_element_type=jnp.float32)

    grid = (_N // _BN,)
    return pl.pallas_call(
        body,
        grid=grid,
        in_specs=[
            pl.BlockSpec((_BN, _D), lambda i: (i, 0)),
            pl.BlockSpec((_D, _D), lambda i: (0, 0)),
        ],
        out_specs=[pl.BlockSpec((_BN, _D), lambda i: (i, 0))],
        out_shape=[jax.ShapeDtypeStruct((_N, _D), jnp.float32)],
    )(x, w1)[0]


def _tc_first(deg0, deg1, h):
    def body(d0_ref, d1_ref, h_ref, dis_ref, g_ref):
        deg = d0_ref[...] + d1_ref[...] + 1.0
        dis = lax.rsqrt(deg)
        dis_ref[...] = dis
        g = h_ref[...] * dis
        g_ref[0, :, :] = g[:, :_HD]
        g_ref[1, :, :] = g[:, _HD:]

    grid = (_N // _BN,)
    return pl.pallas_call(
        body,
        grid=grid,
        in_specs=[
            pl.BlockSpec((_BN, 1), lambda i: (i, 0)),
            pl.BlockSpec((_BN, 1), lambda i: (i, 0)),
            pl.BlockSpec((_BN, _D), lambda i: (i, 0)),
        ],
        out_specs=[
            pl.BlockSpec((_BN, 1), lambda i: (i, 0)),
            pl.BlockSpec((NC, _BN, _HD), lambda i: (0, i, 0)),
        ],
        out_shape=[
            jax.ShapeDtypeStruct((_N, 1), jnp.float32),
            jax.ShapeDtypeStruct((NC, _N, _HD), jnp.float32),
        ],
    )(deg0, deg1, h)


def _tc_mid(dis, s, g, w2, b1):
    def body(dis_ref, s_ref, g_ref, w_ref, b_ref, g2_ref):
        dis = dis_ref[...]
        agg = jnp.concatenate([s_ref[0] + g_ref[0], s_ref[1] + g_ref[1]],
                              axis=1)
        x2 = dis * agg + b_ref[...]
        h2 = jnp.dot(x2, w_ref[...], preferred_element_type=jnp.float32)
        g2 = h2 * dis
        g2_ref[0, :, :] = g2[:, :_HD]
        g2_ref[1, :, :] = g2[:, _HD:]

    grid = (_N // _BN,)
    return pl.pallas_call(
        body,
        grid=grid,
        in_specs=[
            pl.BlockSpec((_BN, 1), lambda i: (i, 0)),
            pl.BlockSpec((NC, _BN, _HD), lambda i: (0, i, 0)),
            pl.BlockSpec((NC, _BN, _HD), lambda i: (0, i, 0)),
            pl.BlockSpec((_D, _D), lambda i: (0, 0)),
            pl.BlockSpec((1, _D), lambda i: (0, 0)),
        ],
        out_specs=[pl.BlockSpec((NC, _BN, _HD), lambda i: (0, i, 0))],
        out_shape=[jax.ShapeDtypeStruct((NC, _N, _HD), jnp.float32)],
    )(dis, s, g, w2, b1)[0]


def _tc_last(dis, s, g2, b2):
    def body(dis_ref, s_ref, g_ref, b_ref, o_ref):
        dis = dis_ref[...]
        agg = jnp.concatenate([s_ref[0] + g_ref[0], s_ref[1] + g_ref[1]],
                              axis=1)
        o_ref[...] = dis * agg + b_ref[...]

    grid = (_N // _BN,)
    return pl.pallas_call(
        body,
        grid=grid,
        in_specs=[
            pl.BlockSpec((_BN, 1), lambda i: (i, 0)),
            pl.BlockSpec((NC, _BN, _HD), lambda i: (0, i, 0)),
            pl.BlockSpec((NC, _BN, _HD), lambda i: (0, i, 0)),
            pl.BlockSpec((1, _D), lambda i: (0, 0)),
        ],
        out_specs=[pl.BlockSpec((_BN, _D), lambda i: (i, 0))],
        out_shape=[jax.ShapeDtypeStruct((_N, _D), jnp.float32)],
    )(dis, s, g2, b2)[0]


def kernel(x, edge_index, W1, b1, W2, b2):
    n, d = x.shape
    e = edge_index.shape[1]
    per = NS * CHUNK
    nch = 16 * (-(-e // (16 * per)))
    e_pad = nch * per
    row = edge_index[0]
    col = edge_index[1]
    pad = e_pad - e
    if pad:
        row = jnp.concatenate([row, jnp.zeros((pad,), jnp.int32)])
        col = jnp.concatenate([col, jnp.full((pad,), n, jnp.int32)])
    r3 = row.reshape(NS, nch, CHUNK)
    c3 = col.reshape(NS, nch, CHUNK)

    z64 = jnp.zeros((128, _HD), jnp.float32)
    z128 = jnp.zeros((128, 128), jnp.float32)
    identp = jnp.broadcast_to(jnp.arange(128, dtype=jnp.int32), (8, 128))
    b1r = b1.reshape(1, d)
    b2r = b2.reshape(1, d)

    h1 = _tc_matmul(x, W1)
    deg0, deg1 = _sc_deg(c3, z128, identp, nch)
    deg0 = deg0.reshape(_DROWS * 128, 1)[:_N]
    deg1 = deg1.reshape(_DROWS * 128, 1)[:_N]
    dis, g1 = _tc_first(deg0, deg1, h1)
    s1 = _sc_aggregate(g1, r3, c3, z64, nch)
    g2 = _tc_mid(dis, s1, g1, W2, b1r)
    s2 = _sc_aggregate(g2, r3, c3, z64, nch)
    return _tc_last(dis, s2, g2, b2r)


# speedup vs baseline: 1.0067x; 1.0061x over previous
import jax
import jax.numpy as jnp
from jax import lax
from jax.experimental import pallas as pl
from jax.experimental.pallas import tpu as pltpu
from jax.experimental.pallas import tpu_sc as plsc

NC = 2
NS = 16
CHUNK = 128

_N = 10000
_D = 128
_HD = _D // 2
_N_ACC = 10240
_ZROWS = _N_ACC // NS
_DROWS = 128


def _mesh():
    return plsc.VectorSubcoreMesh(core_axis_name="c", subcore_axis_name="s")


def _sc_deg(c3, z128, identp, nch):
    half = nch // 2

    def body(c_hbm, z_hbm, id_hbm, out0, out1, cidx, buf, ident, hist, acc):
        ones16 = jnp.full((16,), 1.0, jnp.float32)
        cid = lax.axis_index("c")
        sid = lax.axis_index("s")
        pltpu.sync_copy(z_hbm, buf)
        pltpu.sync_copy(z_hbm, hist)

        @pl.when(sid == 0)
        def _():
            pltpu.sync_copy(buf, acc)

        pltpu.sync_copy(id_hbm, ident)
        pltpu.sync_copy(c_hbm.at[sid, pl.ds(cid * half, half)], cidx)
        plsc.subcore_barrier()

        def step(k, _):
            for j in range(CHUNK // 16):
                c = cidx[k, pl.ds(j * 16, 16)]
                row = lax.shift_right_logical(c, 7)
                lane = lax.bitwise_and(c, 127)
                plsc.addupdate_scatter(hist, [row, lane], ones16)
            return _

        lax.fori_loop(0, half, step, None)
        pltpu.sync_copy(hist, acc.at[ident.at[0]], add=True)
        plsc.subcore_barrier()

        def copy_out(dst):
            pltpu.sync_copy(acc, buf)
            pltpu.sync_copy(buf, dst)

        @pl.when((sid == 0) & (cid == 0))
        def _():
            copy_out(out0)

        @pl.when((sid == 0) & (cid == 1))
        def _():
            copy_out(out1)

    sds = jax.ShapeDtypeStruct((_DROWS, 128), jnp.float32)
    return pl.kernel(
        body,
        out_type=(sds, sds),
        mesh=_mesh(),
        compiler_params=pltpu.CompilerParams(needs_layout_passes=False),
        scratch_types=[
            pltpu.VMEM((half, CHUNK), jnp.int32),
            pltpu.VMEM((_DROWS, 128), jnp.float32),
            pltpu.VMEM((8, CHUNK), jnp.int32),
            pltpu.VMEM((_DROWS, 128), jnp.float32),
            pltpu.VMEM_SHARED((_DROWS, 128), jnp.float32),
        ],
    )(c3, z128, identp)


_NBUF = 4


def _sc_aggregate(g2, r3, c3, z64, nch):
    def body(g_hbm, r_hbm, c_hbm, z_hbm, out,
             ridx, cidx, bufs, acc, gsems, ssems):
        cid = lax.axis_index("c")
        sid = lax.axis_index("s")
        gsrc = g_hbm.at[cid]
        stage_r = pltpu.make_async_copy(r_hbm.at[sid], ridx, gsems.at[0])
        stage_c = pltpu.make_async_copy(c_hbm.at[sid], cidx, gsems.at[1])
        stage_r.start()
        stage_c.start()
        pltpu.sync_copy(z_hbm, bufs.at[0])
        zcopies = [
            pltpu.make_async_copy(
                bufs.at[0, pl.ds(0, 128)],
                acc.at[pl.ds(sid * _ZROWS + j * 128, 128)],
                ssems.at[j % _NBUF])
            for j in range(_ZROWS // 128)
        ]
        for zc in zcopies:
            zc.start()
        for zc in zcopies:
            zc.wait()
        stage_r.wait()
        stage_c.wait()
        plsc.subcore_barrier()

        def gather(k, b):
            return pltpu.make_async_copy(
                gsrc.at[ridx.at[k]], bufs.at[b], gsems.at[b])

        def scatter(k, b):
            return pltpu.make_async_copy(
                bufs.at[b], acc.at[cidx.at[k]], ssems.at[b])

        for b in range(_NBUF):
            gather(b, b).start()

        def step(i, _):
            k = _NBUF * i
            for b in range(_NBUF):
                gather(k + b, b).wait()
                scatter(k + b, b).start(add=True)
            for b in range(_NBUF):
                @pl.when(k + b + _NBUF < nch)
                def _():
                    scatter(k + b, b).wait()
                    gather(k + b + _NBUF, b).start()
            return _

        lax.fori_loop(0, nch // _NBUF, step, None)
        for b in range(_NBUF):
            scatter(nch - _NBUF + b, b).wait()
        plsc.subcore_barrier()

        def ld(j, b):
            return pltpu.make_async_copy(
                acc.at[pl.ds(sid * _ZROWS + j * 128, 128)],
                bufs.at[b, pl.ds(0, 128)], gsems.at[b])

        def st(j, b):
            return pltpu.make_async_copy(
                bufs.at[b, pl.ds(0, 128)],
                out.at[cid, pl.ds(sid * _ZROWS + j * 128, 128)], ssems.at[b])

        nj = _ZROWS // 128
        ld(0, 0).start()
        for j in range(nj):
            b = j % 2
            ld(j, b).wait()
            st(j, b).start()
            if j + 1 < nj:
                if j - 1 >= 0:
                    st(j - 1, 1 - b).wait()
                ld(j + 1, 1 - b).start()
        st(nj - 2, nj % 2).wait()
        st(nj - 1, (nj - 1) % 2).wait()

    return pl.kernel(
        body,
        out_type=jax.ShapeDtypeStruct((NC, _N_ACC, _HD), jnp.float32),
        mesh=_mesh(),
        compiler_params=pltpu.CompilerParams(use_tc_tiling_on_sc=False),
        scratch_types=[
            pltpu.VMEM((nch, CHUNK), jnp.int32),
            pltpu.VMEM((nch, CHUNK), jnp.int32),
            pltpu.VMEM((_NBUF, CHUNK, _HD), jnp.float32),
            pltpu.VMEM_SHARED((_N_ACC, _HD), jnp.float32),
            pltpu.SemaphoreType.DMA((_NBUF,)),
            pltpu.SemaphoreType.DMA((_NBUF,)),
        ],
    )(g2, r3, c3, z64)


_BN = 1000


def _tc_matmul(x, w1):
    def body(x_ref, w_ref, h_ref):
        h_ref[...] = jnp.dot(x_ref[...], w_ref[...],
                             preferred_element_type=jnp.float32)

    grid = (_N // _BN,)
    return pl.pallas_call(
        body,
        grid=grid,
        in_specs=[
            pl.BlockSpec((_BN, _D), lambda i: (i, 0)),
            pl.BlockSpec((_D, _D), lambda i: (0, 0)),
        ],
        out_specs=[pl.BlockSpec((_BN, _D), lambda i: (i, 0))],
        out_shape=[jax.ShapeDtypeStruct((_N, _D), jnp.float32)],
    )(x, w1)[0]


def _tc_first(deg0, deg1, h):
    def body(d0_ref, d1_ref, h_ref, dis_ref, g_ref):
        deg = d0_ref[...] + d1_ref[...] + 1.0
        dis = lax.rsqrt(deg)
        dis_ref[...] = dis
        g = h_ref[...] * dis
        g_ref[0, :, :] = g[:, :_HD]
        g_ref[1, :, :] = g[:, _HD:]

    grid = (_N // _BN,)
    return pl.pallas_call(
        body,
        grid=grid,
        in_specs=[
            pl.BlockSpec((_BN, 1), lambda i: (i, 0)),
            pl.BlockSpec((_BN, 1), lambda i: (i, 0)),
            pl.BlockSpec((_BN, _D), lambda i: (i, 0)),
        ],
        out_specs=[
            pl.BlockSpec((_BN, 1), lambda i: (i, 0)),
            pl.BlockSpec((NC, _BN, _HD), lambda i: (0, i, 0)),
        ],
        out_shape=[
            jax.ShapeDtypeStruct((_N, 1), jnp.float32),
            jax.ShapeDtypeStruct((NC, _N, _HD), jnp.float32),
        ],
    )(deg0, deg1, h)


def _tc_mid(dis, s, g, w2, b1):
    def body(dis_ref, s_ref, g_ref, w_ref, b_ref, g2_ref):
        dis = dis_ref[...]
        agg = jnp.concatenate([s_ref[0] + g_ref[0], s_ref[1] + g_ref[1]],
                              axis=1)
        x2 = dis * agg + b_ref[...]
        h2 = jnp.dot(x2, w_ref[...], preferred_element_type=jnp.float32)
        g2 = h2 * dis
        g2_ref[0, :, :] = g2[:, :_HD]
        g2_ref[1, :, :] = g2[:, _HD:]

    grid = (_N // _BN,)
    return pl.pallas_call(
        body,
        grid=grid,
        in_specs=[
            pl.BlockSpec((_BN, 1), lambda i: (i, 0)),
            pl.BlockSpec((NC, _BN, _HD), lambda i: (0, i, 0)),
            pl.BlockSpec((NC, _BN, _HD), lambda i: (0, i, 0)),
            pl.BlockSpec((_D, _D), lambda i: (0, 0)),
            pl.BlockSpec((1, _D), lambda i: (0, 0)),
        ],
        out_specs=[pl.BlockSpec((NC, _BN, _HD), lambda i: (0, i, 0))],
        out_shape=[jax.ShapeDtypeStruct((NC, _N, _HD), jnp.float32)],
    )(dis, s, g, w2, b1)[0]


def _tc_last(dis, s, g2, b2):
    def body(dis_ref, s_ref, g_ref, b_ref, o_ref):
        dis = dis_ref[...]
        agg = jnp.concatenate([s_ref[0] + g_ref[0], s_ref[1] + g_ref[1]],
                              axis=1)
        o_ref[...] = dis * agg + b_ref[...]

    grid = (_N // _BN,)
    return pl.pallas_call(
        body,
        grid=grid,
        in_specs=[
            pl.BlockSpec((_BN, 1), lambda i: (i, 0)),
            pl.BlockSpec((NC, _BN, _HD), lambda i: (0, i, 0)),
            pl.BlockSpec((NC, _BN, _HD), lambda i: (0, i, 0)),
            pl.BlockSpec((1, _D), lambda i: (0, 0)),
        ],
        out_specs=[pl.BlockSpec((_BN, _D), lambda i: (i, 0))],
        out_shape=[jax.ShapeDtypeStruct((_N, _D), jnp.float32)],
    )(dis, s, g2, b2)[0]


def kernel(x, edge_index, W1, b1, W2, b2):
    n, d = x.shape
    e = edge_index.shape[1]
    per = NS * CHUNK
    nch = 16 * (-(-e // (16 * per)))
    e_pad = nch * per
    row = edge_index[0]
    col = edge_index[1]
    pad = e_pad - e
    if pad:
        row = jnp.concatenate([row, jnp.zeros((pad,), jnp.int32)])
        col = jnp.concatenate([col, jnp.full((pad,), n, jnp.int32)])
    r3 = row.reshape(NS, nch, CHUNK)
    c3 = col.reshape(NS, nch, CHUNK)

    z64 = jnp.zeros((128, _HD), jnp.float32)
    z128 = jnp.zeros((128, 128), jnp.float32)
    identp = jnp.broadcast_to(jnp.arange(128, dtype=jnp.int32), (8, 128))
    b1r = b1.reshape(1, d)
    b2r = b2.reshape(1, d)

    h1 = _tc_matmul(x, W1)
    deg0, deg1 = _sc_deg(c3, z128, identp, nch)
    deg0 = deg0.reshape(_DROWS * 128, 1)[:_N]
    deg1 = deg1.reshape(_DROWS * 128, 1)[:_N]
    dis, g1 = _tc_first(deg0, deg1, h1)
    s1 = _sc_aggregate(g1, r3, c3, z64, nch)
    g2 = _tc_mid(dis, s1, g1, W2, b1r)
    s2 = _sc_aggregate(g2, r3, c3, z64, nch)
    return _tc_last(dis, s2, g2, b2r)
